# baseline (speedup 1.0000x reference)
import jax
import jax.numpy as jnp
from jax import lax
from jax.experimental import pallas as pl
from jax.experimental.pallas import tpu as pltpu
from jax.experimental.pallas import tpu_sc as plsc

B = 16384
EMB = 64
SUB = 8
NC, NS = 2, 16
NW = NC * NS
BPW = B // NW
CH = 128
NCHT = BPW // CH
CB = 2048


def _gather_body(user_t, movie_t, ids3, out, idx_v, slab_v, rows_v, sem):
    wid = lax.axis_index("s") * NC + lax.axis_index("c")
    base = wid * BPW
    for t in range(3):
        pltpu.sync_copy(ids3.at[t, pl.ds(wid * NCHT, NCHT)],
                        idx_v.at[pl.ds(t * NCHT, NCHT)])

    W = 32
    for t, table in ((0, user_t), (1, movie_t), (2, movie_t)):
        def _wave(w, _, t=t, table=table):
            k = t * NCHT + lax.shift_right_logical(w, 2)
            off = lax.bitwise_and(w, 3) * W

            def _fire(g, _):
                vec = idx_v[k, pl.ds(off + g * 16, 16)]
                for u in range(16):
                    slab8 = pl.multiple_of(
                        lax.bitwise_and(vec[u], ~jnp.int32(7)), SUB)
                    pltpu.async_copy(table.at[pl.ds(slab8, SUB)],
                                     slab_v.at[pl.ds((g * 16 + u) * SUB, SUB)],
                                     sem)
                return _

            lax.fori_loop(0, W // 16, _fire, None)
            pltpu.make_async_copy(out.at[t, pl.ds(0, W * SUB)], slab_v,
                                  sem).wait()

            def _extract(g, _):
                vec = idx_v[k, pl.ds(off + g * 16, 16)]
                for u in range(16):
                    sub = lax.bitwise_and(vec[u], 7)
                    src_r = (g * 16 + u) * SUB + sub
                    dst_r = w * W + g * 16 + u
                    for c in range(EMB // 16):
                        v = slab_v[src_r, pl.ds(c * 16, 16)]
                        rows_v[dst_r, pl.ds(c * 16, 16)] = v
                return _

            lax.fori_loop(0, W // 16, _extract, None)
            return _

        lax.fori_loop(0, BPW // W, _wave, None)
        pltpu.sync_copy(rows_v, out.at[t, pl.ds(base, BPW)])


def _sc_gather(user_t, movie_t, ids3):
    mesh = plsc.VectorSubcoreMesh(core_axis_name="c", subcore_axis_name="s")
    return pl.kernel(
        _gather_body,
        mesh=mesh,
        out_type=jax.ShapeDtypeStruct((3, B, EMB), jnp.float32),
        scratch_types=[
            pltpu.VMEM((3 * NCHT, CH), jnp.int32),
            pltpu.VMEM((32 * SUB, EMB), jnp.float32),
            pltpu.VMEM((BPW, EMB), jnp.float32),
            pltpu.SemaphoreType.DMA,
        ],
    )(user_t, movie_t, ids3)


def _mlp_body(emb_ref, w1_ref, b1_ref, w2_ref, b2_ref, out_ref):
    e = emb_ref[0]
    h = jnp.dot(e, w1_ref[0], preferred_element_type=jnp.float32) + b1_ref[0]
    h = jnp.maximum(h, 0.0)
    out_ref[0] = (jnp.dot(h, w2_ref[0], preferred_element_type=jnp.float32)
                  + b2_ref[0])


def _tc_mlp(emb3, w1s, b1s, w2s, b2s):
    return pl.pallas_call(
        _mlp_body,
        grid=(3, B // CB),
        in_specs=[
            pl.BlockSpec((1, CB, EMB), lambda t, i: (t, i, 0)),
            pl.BlockSpec((1, EMB, 64), lambda t, i: (jnp.minimum(t, 1), 0, 0)),
            pl.BlockSpec((1, 1, 64), lambda t, i: (jnp.minimum(t, 1), 0, 0)),
            pl.BlockSpec((1, 64, 32), lambda t, i: (jnp.minimum(t, 1), 0, 0)),
            pl.BlockSpec((1, 1, 32), lambda t, i: (jnp.minimum(t, 1), 0, 0)),
        ],
        out_specs=pl.BlockSpec((1, CB, 32), lambda t, i: (t, i, 0)),
        out_shape=jax.ShapeDtypeStruct((3, B, 32), jnp.float32),
    )(emb3, w1s, b1s, w2s, b2s)


def kernel(user_ids, pos_movie_ids, neg_movie_ids, user_table, movie_table,
           uW1, ub1, uW2, ub2, mW1, mb1, mW2, mb2):
    ids3 = jnp.stack([user_ids, pos_movie_ids, neg_movie_ids]).astype(jnp.int32)
    ids3 = ids3.reshape(3, B // CH, CH)
    emb3 = _sc_gather(user_table, movie_table, ids3)
    w1s = jnp.stack([uW1, mW1])
    b1s = jnp.stack([ub1, mb1]).reshape(2, 1, 64)
    w2s = jnp.stack([uW2, mW2])
    b2s = jnp.stack([ub2, mb2]).reshape(2, 1, 32)
    out3 = _tc_mlp(emb3, w1s, b1s, w2s, b2s)
    return out3[0], out3[1], out3[2]


# speedup vs baseline: 1.0670x; 1.0670x over previous
import jax
import jax.numpy as jnp
from jax import lax
from jax.experimental import pallas as pl
from jax.experimental.pallas import tpu as pltpu
from jax.experimental.pallas import tpu_sc as plsc

B = 16384
EMB = 64
SUB = 8
NC, NS = 2, 16
NW = NC * NS
BPW = B // NW
CH = 128
NCHT = BPW // CH
UW = 128
NUH = 500000
CB = 2048
RB = 4000


def _compact_body(lo_ref, hi_ref, out_ref):
    out_ref[...] = jnp.concatenate([lo_ref[...], hi_ref[...]], axis=1)


def _tc_compact(table):
    n = table.shape[0]
    nh = n // 2
    return pl.pallas_call(
        _compact_body,
        grid=(nh // RB,),
        in_specs=[
            pl.BlockSpec((RB, EMB), lambda i: (i, 0)),
            pl.BlockSpec((RB, EMB), lambda i, nb=nh // RB: (i + nb, 0)),
        ],
        out_specs=pl.BlockSpec((RB, 2 * EMB), lambda i: (i, 0)),
        out_shape=jax.ShapeDtypeStruct((nh, 2 * EMB), jnp.float32),
    )(table, table)


def _gather_body(user_c, movie_t3, ids3, out, idx_v, ubuf, rows_v, sem, sem2):
    wid = lax.axis_index("s") * NC + lax.axis_index("c")
    base = wid * BPW
    for t in range(3):
        pltpu.sync_copy(ids3.at[t, pl.ds(wid * NCHT, NCHT)],
                        idx_v.at[pl.ds(t * NCHT, NCHT)])

    def _uwave(w, _):
        k = lax.shift_right_logical(w, 0)

        def _ufire(g, _):
            vec = idx_v[w, pl.ds(g * 16, 16)]
            for u in range(16):
                rid = vec[u]
                half = jnp.where(rid < NUH, rid, rid - NUH)
                pltpu.async_copy(user_c.at[pl.ds(half, 1)],
                                 ubuf.at[pl.ds(g * 16 + u, 1)], sem)
            return _

        lax.fori_loop(0, UW // 16, _ufire, None)
        pltpu.make_async_copy(user_c.at[pl.ds(0, UW)], ubuf, sem).wait()

        def _uext(g, _):
            vec = idx_v[w, pl.ds(g * 16, 16)]
            for u in range(16):
                rid = vec[u]
                off = jnp.where(rid < NUH, 0, EMB)
                dst_r = w * UW + g * 16 + u
                for c in range(EMB // 16):
                    v = ubuf[g * 16 + u, pl.ds(off + c * 16, 16)]
                    rows_v[dst_r, pl.ds(c * 16, 16)] = v
            return _

        lax.fori_loop(0, UW // 16, _uext, None)
        return _

    lax.fori_loop(0, BPW // UW, _uwave, None)
    pltpu.sync_copy(rows_v, out.at[0, pl.ds(base, BPW)])

    for t in (1, 2):
        for ch in range(NCHT):
            def _mfire(g, _, t=t, ch=ch):
                vec = idx_v[t * NCHT + ch, pl.ds(g * 16, 16)]
                j0 = ch * CH + g * 16
                for u in range(16):
                    rid = vec[u]
                    slab = lax.shift_right_logical(rid, 3)
                    sub = lax.bitwise_and(rid, 7)
                    pltpu.async_copy(movie_t3.at[pl.ds(slab, 1), sub],
                                     rows_v.at[pl.ds(j0 + u, 1)], sem2)
                return _

            lax.fori_loop(0, CH // 16, _mfire, None)
        pltpu.make_async_copy(out.at[t, pl.ds(base, BPW)], rows_v, sem2).wait()
        pltpu.sync_copy(rows_v, out.at[t, pl.ds(base, BPW)])


def _sc_gather(user_c, movie_t3, ids3):
    mesh = plsc.VectorSubcoreMesh(core_axis_name="c", subcore_axis_name="s")
    return pl.kernel(
        _gather_body,
        mesh=mesh,
        out_type=jax.ShapeDtypeStruct((3, B, EMB), jnp.float32),
        scratch_types=[
            pltpu.VMEM((3 * NCHT, CH), jnp.int32),
            pltpu.VMEM((UW, 2 * EMB), jnp.float32),
            pltpu.VMEM((BPW, EMB), jnp.float32),
            pltpu.SemaphoreType.DMA,
            pltpu.SemaphoreType.DMA,
        ],
    )(user_c, movie_t3, ids3)


def _mlp_body(emb_ref, w1_ref, b1_ref, w2_ref, b2_ref, out_ref):
    e = emb_ref[0]
    h = jnp.dot(e, w1_ref[0], preferred_element_type=jnp.float32) + b1_ref[0]
    h = jnp.maximum(h, 0.0)
    out_ref[0] = (jnp.dot(h, w2_ref[0], preferred_element_type=jnp.float32)
                  + b2_ref[0])


def _tc_mlp(emb3, w1s, b1s, w2s, b2s):
    return pl.pallas_call(
        _mlp_body,
        grid=(3, B // CB),
        in_specs=[
            pl.BlockSpec((1, CB, EMB), lambda t, i: (t, i, 0)),
            pl.BlockSpec((1, EMB, 64), lambda t, i: (jnp.minimum(t, 1), 0, 0)),
            pl.BlockSpec((1, 1, 64), lambda t, i: (jnp.minimum(t, 1), 0, 0)),
            pl.BlockSpec((1, 64, 32), lambda t, i: (jnp.minimum(t, 1), 0, 0)),
            pl.BlockSpec((1, 1, 32), lambda t, i: (jnp.minimum(t, 1), 0, 0)),
        ],
        out_specs=pl.BlockSpec((1, CB, 32), lambda t, i: (t, i, 0)),
        out_shape=jax.ShapeDtypeStruct((3, B, 32), jnp.float32),
    )(emb3, w1s, b1s, w2s, b2s)


def kernel(user_ids, pos_movie_ids, neg_movie_ids, user_table, movie_table,
           uW1, ub1, uW2, ub2, mW1, mb1, mW2, mb2):
    ids3 = jnp.stack([user_ids, pos_movie_ids, neg_movie_ids]).astype(jnp.int32)
    ids3 = ids3.reshape(3, B // CH, CH)
    user_c = _tc_compact(user_table)
    movie_t3 = movie_table.reshape(1000000 // SUB, SUB, EMB)
    emb3 = _sc_gather(user_c, movie_t3, ids3)
    w1s = jnp.stack([uW1, mW1])
    b1s = jnp.stack([ub1, mb1]).reshape(2, 1, 64)
    w2s = jnp.stack([uW2, mW2])
    b2s = jnp.stack([ub2, mb2]).reshape(2, 1, 32)
    out3 = _tc_mlp(emb3, w1s, b1s, w2s, b2s)
    return out3[0], out3[1], out3[2]


# speedup vs baseline: 1.0842x; 1.0161x over previous
import jax
import jax.numpy as jnp
from jax import lax
from jax.experimental import pallas as pl
from jax.experimental.pallas import tpu as pltpu
from jax.experimental.pallas import tpu_sc as plsc

B = 16384
EMB = 64
SUB = 8
NC, NS = 2, 16
NW = NC * NS
BPW = B // NW
CH = 128
NCHT = BPW // CH
UW = 128
NUH = 500000
CB = 2048
RB = 10000


def _compact_body(lo_ref, hi_ref, out_ref):
    out_ref[:, :EMB] = lo_ref[...]
    out_ref[:, EMB:] = hi_ref[...]


def _tc_compact(table):
    n = table.shape[0]
    nh = n // 2
    return pl.pallas_call(
        _compact_body,
        grid=(nh // RB,),
        in_specs=[
            pl.BlockSpec((RB, EMB), lambda i: (i, 0)),
            pl.BlockSpec((RB, EMB), lambda i, nb=nh // RB: (i + nb, 0)),
        ],
        out_specs=pl.BlockSpec((RB, 2 * EMB), lambda i: (i, 0)),
        out_shape=jax.ShapeDtypeStruct((nh, 2 * EMB), jnp.float32),
    )(table, table)


def _gather_body(user_c, movie_t3, ids3, out, idx_v, ubuf, rows_v, sem, sem2):
    wid = lax.axis_index("s") * NC + lax.axis_index("c")
    base = wid * BPW
    for t in range(3):
        pltpu.sync_copy(ids3.at[t, pl.ds(wid * NCHT, NCHT)],
                        idx_v.at[pl.ds(t * NCHT, NCHT)])

    def _uwave(w, _):
        k = lax.shift_right_logical(w, 0)

        def _ufire(g, _):
            vec = idx_v[w, pl.ds(g * 16, 16)]
            for u in range(16):
                rid = vec[u]
                half = jnp.where(rid < NUH, rid, rid - NUH)
                pltpu.async_copy(user_c.at[pl.ds(half, 1)],
                                 ubuf.at[pl.ds(g * 16 + u, 1)], sem)
            return _

        lax.fori_loop(0, UW // 16, _ufire, None)
        pltpu.make_async_copy(user_c.at[pl.ds(0, UW)], ubuf, sem).wait()

        def _uext(g, _):
            vec = idx_v[w, pl.ds(g * 16, 16)]
            for u in range(16):
                rid = vec[u]
                off = jnp.where(rid < NUH, 0, EMB)
                dst_r = w * UW + g * 16 + u
                for c in range(EMB // 16):
                    v = ubuf[g * 16 + u, pl.ds(off + c * 16, 16)]
                    rows_v[dst_r, pl.ds(c * 16, 16)] = v
            return _

        lax.fori_loop(0, UW // 16, _uext, None)
        return _

    lax.fori_loop(0, BPW // UW, _uwave, None)
    pltpu.sync_copy(rows_v, out.at[0, pl.ds(base, BPW)])

    for t in (1, 2):
        for ch in range(NCHT):
            def _mfire(g, _, t=t, ch=ch):
                vec = idx_v[t * NCHT + ch, pl.ds(g * 16, 16)]
                j0 = ch * CH + g * 16
                for u in range(16):
                    rid = vec[u]
                    slab = lax.shift_right_logical(rid, 3)
                    sub = lax.bitwise_and(rid, 7)
                    pltpu.async_copy(movie_t3.at[pl.ds(slab, 1), sub],
                                     rows_v.at[pl.ds(j0 + u, 1)], sem2)
                return _

            lax.fori_loop(0, CH // 16, _mfire, None)
        pltpu.make_async_copy(out.at[t, pl.ds(base, BPW)], rows_v, sem2).wait()
        pltpu.sync_copy(rows_v, out.at[t, pl.ds(base, BPW)])


def _sc_gather(user_c, movie_t3, ids3):
    mesh = plsc.VectorSubcoreMesh(core_axis_name="c", subcore_axis_name="s")
    return pl.kernel(
        _gather_body,
        mesh=mesh,
        out_type=jax.ShapeDtypeStruct((3, B, EMB), jnp.float32),
        scratch_types=[
            pltpu.VMEM((3 * NCHT, CH), jnp.int32),
            pltpu.VMEM((UW, 2 * EMB), jnp.float32),
            pltpu.VMEM((BPW, EMB), jnp.float32),
            pltpu.SemaphoreType.DMA,
            pltpu.SemaphoreType.DMA,
        ],
    )(user_c, movie_t3, ids3)


def _mlp_body(emb_ref, w1_ref, b1_ref, w2_ref, b2_ref, out_ref):
    e = emb_ref[0]
    h = jnp.dot(e, w1_ref[0], preferred_element_type=jnp.float32) + b1_ref[0]
    h = jnp.maximum(h, 0.0)
    out_ref[0] = (jnp.dot(h, w2_ref[0], preferred_element_type=jnp.float32)
                  + b2_ref[0])


def _tc_mlp(emb3, w1s, b1s, w2s, b2s):
    return pl.pallas_call(
        _mlp_body,
        grid=(3, B // CB),
        in_specs=[
            pl.BlockSpec((1, CB, EMB), lambda t, i: (t, i, 0)),
            pl.BlockSpec((1, EMB, 64), lambda t, i: (jnp.minimum(t, 1), 0, 0)),
            pl.BlockSpec((1, 1, 64), lambda t, i: (jnp.minimum(t, 1), 0, 0)),
            pl.BlockSpec((1, 64, 32), lambda t, i: (jnp.minimum(t, 1), 0, 0)),
            pl.BlockSpec((1, 1, 32), lambda t, i: (jnp.minimum(t, 1), 0, 0)),
        ],
        out_specs=pl.BlockSpec((1, CB, 32), lambda t, i: (t, i, 0)),
        out_shape=jax.ShapeDtypeStruct((3, B, 32), jnp.float32),
    )(emb3, w1s, b1s, w2s, b2s)


def kernel(user_ids, pos_movie_ids, neg_movie_ids, user_table, movie_table,
           uW1, ub1, uW2, ub2, mW1, mb1, mW2, mb2):
    ids3 = jnp.stack([user_ids, pos_movie_ids, neg_movie_ids]).astype(jnp.int32)
    ids3 = ids3.reshape(3, B // CH, CH)
    user_c = _tc_compact(user_table)
    movie_t3 = movie_table.reshape(1000000 // SUB, SUB, EMB)
    emb3 = _sc_gather(user_c, movie_t3, ids3)
    w1s = jnp.stack([uW1, mW1])
    b1s = jnp.stack([ub1, mb1]).reshape(2, 1, 64)
    w2s = jnp.stack([uW2, mW2])
    b2s = jnp.stack([ub2, mb2]).reshape(2, 1, 32)
    out3 = _tc_mlp(emb3, w1s, b1s, w2s, b2s)
    return out3[0], out3[1], out3[2]


# speedup vs baseline: 1.6949x; 1.5633x over previous
import jax
import jax.numpy as jnp
from jax import lax
from jax.experimental import pallas as pl
from jax.experimental.pallas import tpu as pltpu
from jax.experimental.pallas import tpu_sc as plsc

B = 16384
EMB = 64
SUB = 8
NC, NS = 2, 16
NW = NC * NS
BPW = B // NW
CH = 128
NCHT = BPW // CH
CB = 2048


def _gather_body(user_t3, movie_t3, ids3, out, idx_v, rows_v, sem):
    wid = lax.axis_index("s") * NC + lax.axis_index("c")
    base = wid * BPW
    for t in range(3):
        pltpu.sync_copy(ids3.at[t, pl.ds(wid * NCHT, NCHT)],
                        idx_v.at[pl.ds(t * NCHT, NCHT)])

    for t, table in ((0, user_t3), (1, movie_t3), (2, movie_t3)):
        for ch in range(NCHT):
            def _group(g, _, t=t, table=table, ch=ch):
                vec = idx_v[t * NCHT + ch, pl.ds(g * 16, 16)]
                j0 = ch * CH + g * 16
                for u in range(16):
                    rid = vec[u]
                    slab = lax.shift_right_logical(rid, 3)
                    sub = lax.bitwise_and(rid, 7)
                    pltpu.async_copy(table.at[pl.ds(slab, 1), sub],
                                     rows_v.at[pl.ds(j0 + u, 1)], sem)
                return _

            lax.fori_loop(0, CH // 16, _group, None)
        pltpu.make_async_copy(out.at[t, pl.ds(base, BPW)], rows_v, sem).wait()
        pltpu.sync_copy(rows_v, out.at[t, pl.ds(base, BPW)])


def _sc_gather(user_t3, movie_t3, ids3):
    mesh = plsc.VectorSubcoreMesh(core_axis_name="c", subcore_axis_name="s")
    return pl.kernel(
        _gather_body,
        mesh=mesh,
        out_type=jax.ShapeDtypeStruct((3, B, EMB), jnp.float32),
        scratch_types=[
            pltpu.VMEM((3 * NCHT, CH), jnp.int32),
            pltpu.VMEM((BPW, EMB), jnp.float32),
            pltpu.SemaphoreType.DMA,
        ],
    )(user_t3, movie_t3, ids3)


def _mlp_body(emb_ref, w1_ref, b1_ref, w2_ref, b2_ref, ou_ref, op_ref, on_ref):
    t = pl.program_id(1)
    e = emb_ref[0]
    h = jnp.dot(e, w1_ref[0], preferred_element_type=jnp.float32) + b1_ref[0]
    h = jnp.maximum(h, 0.0)
    o = (jnp.dot(h, w2_ref[0], preferred_element_type=jnp.float32)
         + b2_ref[0])

    @pl.when(t == 0)
    def _():
        ou_ref[...] = o

    @pl.when(t == 1)
    def _():
        op_ref[...] = o

    @pl.when(t == 2)
    def _():
        on_ref[...] = o


def _tc_mlp(emb3, w1s, b1s, w2s, b2s):
    ovec = jax.ShapeDtypeStruct((B, 32), jnp.float32)
    return pl.pallas_call(
        _mlp_body,
        grid=(B // CB, 3),
        in_specs=[
            pl.BlockSpec((1, CB, EMB), lambda i, t: (t, i, 0)),
            pl.BlockSpec((1, EMB, 64), lambda i, t: (jnp.minimum(t, 1), 0, 0)),
            pl.BlockSpec((1, 1, 64), lambda i, t: (jnp.minimum(t, 1), 0, 0)),
            pl.BlockSpec((1, 64, 32), lambda i, t: (jnp.minimum(t, 1), 0, 0)),
            pl.BlockSpec((1, 1, 32), lambda i, t: (jnp.minimum(t, 1), 0, 0)),
        ],
        out_specs=[
            pl.BlockSpec((CB, 32), lambda i, t: (i, 0)),
            pl.BlockSpec((CB, 32), lambda i, t: (i, 0)),
            pl.BlockSpec((CB, 32), lambda i, t: (i, 0)),
        ],
        out_shape=[ovec, ovec, ovec],
    )(emb3, w1s, b1s, w2s, b2s)


def kernel(user_ids, pos_movie_ids, neg_movie_ids, user_table, movie_table,
           uW1, ub1, uW2, ub2, mW1, mb1, mW2, mb2):
    ids3 = jnp.stack([user_ids, pos_movie_ids, neg_movie_ids]).astype(jnp.int32)
    ids3 = ids3.reshape(3, B // CH, CH)
    user_t3 = user_table.reshape(1000000 // SUB, SUB, EMB)
    movie_t3 = movie_table.reshape(1000000 // SUB, SUB, EMB)
    emb3 = _sc_gather(user_t3, movie_t3, ids3)
    w1s = jnp.stack([uW1, mW1])
    b1s = jnp.stack([ub1, mb1]).reshape(2, 1, 64)
    w2s = jnp.stack([uW2, mW2])
    b2s = jnp.stack([ub2, mb2]).reshape(2, 1, 32)
    return _tc_mlp(emb3, w1s, b1s, w2s, b2s)
